# baseline (speedup 1.0000x reference)
import jax
import jax.numpy as jnp
from jax.experimental import pallas as pl
from jax.experimental.pallas import tpu as pltpu

_BM = 200
_NS = 5


def _sgc_body(w_ref, b_ref, x_hbm, adj_hbm, o_ref,
              xw_ref, x_ref, buf, sems, x_sem):
    n = x_ref.shape[0]
    nb = n // _BM

    def adj_copy(blk):
        return pltpu.make_async_copy(
            adj_hbm.at[pl.ds(blk * _BM, _BM), :],
            buf.at[blk % _NS],
            sems.at[blk % _NS],
        )

    x_copy = pltpu.make_async_copy(x_hbm, x_ref, x_sem)
    x_copy.start()
    for j in range(_NS - 1):
        adj_copy(j).start()

    x_copy.wait()
    xw_ref[...] = jax.lax.dot_general(
        x_ref[...], w_ref[...],
        (((1,), (1,)), ((), ())),
        preferred_element_type=jnp.float32,
    )
    bias = b_ref[...]

    for blk in range(nb):
        adj_copy(blk).wait()
        nxt = blk + _NS - 1
        if nxt < nb:
            adj_copy(nxt).start()
        o_ref[blk * _BM:(blk + 1) * _BM, :] = (
            jnp.dot(buf[blk % _NS], xw_ref[...],
                    preferred_element_type=jnp.float32)
            + bias
        )


def kernel(x, adj, W, b):
    n, nfeat = x.shape
    nclass = W.shape[0]
    b2 = b.reshape(1, nclass)
    out = pl.pallas_call(
        _sgc_body,
        in_specs=[
            pl.BlockSpec(memory_space=pltpu.MemorySpace.VMEM),
            pl.BlockSpec(memory_space=pltpu.MemorySpace.VMEM),
            pl.BlockSpec(memory_space=pltpu.MemorySpace.HBM),
            pl.BlockSpec(memory_space=pltpu.MemorySpace.HBM),
        ],
        out_specs=pl.BlockSpec(memory_space=pltpu.MemorySpace.VMEM),
        out_shape=jax.ShapeDtypeStruct((n, nclass), jnp.float32),
        scratch_shapes=[
            pltpu.VMEM((n, nclass), jnp.float32),
            pltpu.VMEM((n, nfeat), jnp.float32),
            pltpu.VMEM((_NS, _BM, n), jnp.float32),
            pltpu.SemaphoreType.DMA((_NS,)),
            pltpu.SemaphoreType.DMA,
        ],
    )(W, b2, x, adj)
    return out


# speedup vs baseline: 1.0300x; 1.0300x over previous
import jax
import jax.numpy as jnp
from jax.experimental import pallas as pl
from jax.experimental.pallas import tpu as pltpu


def _sgc_kernel(adj_ref, x_ref, w_ref, b_ref, o_ref):
    xw = jax.lax.dot_general(
        x_ref[...], w_ref[...],
        (((1,), (1,)), ((), ())),
        preferred_element_type=jnp.float32,
    )
    o_ref[...] = (
        jnp.dot(adj_ref[...].astype(jnp.bfloat16),
                xw.astype(jnp.bfloat16),
                preferred_element_type=jnp.float32)
        + b_ref[...]
    )


def kernel(x, adj, W, b):
    n, nfeat = x.shape
    nclass = W.shape[0]
    b2 = b.reshape(1, nclass)

    bm = 400
    grid = (n // bm,)
    out = pl.pallas_call(
        _sgc_kernel,
        grid=grid,
        in_specs=[
            pl.BlockSpec((bm, n), lambda i: (i, 0)),
            pl.BlockSpec((n, nfeat), lambda i: (0, 0)),
            pl.BlockSpec((nclass, nfeat), lambda i: (0, 0)),
            pl.BlockSpec((1, nclass), lambda i: (0, 0)),
        ],
        out_specs=pl.BlockSpec((bm, nclass), lambda i: (i, 0)),
        out_shape=jax.ShapeDtypeStruct((n, nclass), jnp.float32),
        compiler_params=pltpu.CompilerParams(
            dimension_semantics=("parallel",),
        ),
    )(adj, x, W, b2)
    return out
